# Initial kernel scaffold; baseline (speedup 1.0000x reference)
import jax
import jax.numpy as jnp
from jax.experimental import pallas as pl


def kernel(x, edge_index, params):
    raise NotImplementedError("write your pallas kernel here")


# speedup vs baseline: 24.1165x; 24.1165x over previous
import functools

import jax
import jax.numpy as jnp
from jax import lax
from jax.experimental import pallas as pl
from jax.experimental.pallas import tpu as pltpu
from jax.experimental.pallas import tpu_sc as plsc

N_NODES = 50000
NODES_PAD = 50176
TRASH = 50048
ROWS_PER_TILE = 3200
ACC_ROWS = ROWS_PER_TILE * 16
CHUNK = 128
E_PAD = 802816
N_CHUNKS = E_PAD // CHUNK
NCORES = 2
NSUBS = 16
BLK = 512
GRID = NODES_PAD // BLK

LAYER_DIMS = [(16, 16, 5), (16, 32, 4), (32, 64, 3), (64, 152, 2)]


def _sc_mesh():
    return plsc.VectorSubcoreMesh(core_axis_name="c", subcore_axis_name="s")


def _fill_rows(ref, value, width):
    vec = jnp.full((16,), value, jnp.float32)

    def body(i, carry):
        for j in range(width // 16):
            ref[i, pl.ds(j * 16, 16)] = vec
        return carry

    lax.fori_loop(0, CHUNK, body, 0)


def _zero_acc(acc, zbuf, sid):
    def body(t, carry):
        pltpu.sync_copy(
            zbuf, acc.at[pl.ds(sid * ROWS_PER_TILE + t * CHUNK, CHUNK)])
        return carry

    lax.fori_loop(0, ROWS_PER_TILE // CHUNK, body, 0)


@functools.lru_cache(maxsize=None)
def _make_spmv(wb, feature_split):
    if feature_split:
        chunks_per_tile = N_CHUNKS // NSUBS
    else:
        chunks_per_tile = N_CHUNKS // (NCORES * NSUBS)
    group = 4
    iters = chunks_per_tile // group
    assert iters * group == chunks_per_tile

    @functools.partial(
        pl.kernel,
        out_type=jax.ShapeDtypeStruct((NCORES, ACC_ROWS, wb), jnp.float32),
        mesh=_sc_mesh(),
        compiler_params=pltpu.CompilerParams(use_tc_tiling_on_sc=False),
        scratch_types=[
            pltpu.VMEM_SHARED((ACC_ROWS, wb), jnp.float32),
            pltpu.VMEM((CHUNK, wb), jnp.float32),
            pltpu.VMEM((group, CHUNK), jnp.int32),
            pltpu.VMEM((group, CHUNK), jnp.int32),
            pltpu.VMEM((group, CHUNK, wb), jnp.float32),
            pltpu.SemaphoreType.DMA,
        ],
    )
    def spmv(s_tab, gidx_hbm, sidx_hbm, out_hbm, acc, zbuf, gi, si, data, sem):
        cid = lax.axis_index("c")
        sid = lax.axis_index("s")
        _fill_rows(zbuf, 0.0, wb)
        _zero_acc(acc, zbuf, sid)
        plsc.subcore_barrier()

        if feature_split:
            chunk0 = sid * chunks_per_tile
        else:
            chunk0 = (cid * NSUBS + sid) * chunks_per_tile

        def body(it, carry):
            base = chunk0 + it * group
            pltpu.sync_copy(gidx_hbm.at[pl.ds(base, group)], gi)
            pltpu.sync_copy(sidx_hbm.at[pl.ds(base, group)], si)
            if feature_split:
                off = cid * NODES_PAD
                for g in range(group):
                    for j in range(CHUNK // 16):
                        v = gi[g, pl.ds(j * 16, 16)]
                        gi[g, pl.ds(j * 16, 16)] = v + off
            copies = [
                pltpu.async_copy(s_tab.at[gi.at[g]], data.at[g], sem)
                for g in range(group)
            ]
            for cp in copies:
                cp.wait()
            for g in range(group):
                pltpu.sync_copy(data.at[g], acc.at[si.at[g]], add=True)
            return carry

        lax.fori_loop(0, iters, body, 0)
        plsc.subcore_barrier()
        pltpu.sync_copy(
            acc.at[pl.ds(sid * ROWS_PER_TILE, ROWS_PER_TILE)],
            out_hbm.at[cid, pl.ds(sid * ROWS_PER_TILE, ROWS_PER_TILE)])

    return spmv


@functools.lru_cache(maxsize=None)
def _make_degree():
    wb = 16
    chunks_per_tile = N_CHUNKS // (NCORES * NSUBS)
    group = 4
    iters = chunks_per_tile // group

    @functools.partial(
        pl.kernel,
        out_type=jax.ShapeDtypeStruct((NCORES, ACC_ROWS, wb), jnp.float32),
        mesh=_sc_mesh(),
        compiler_params=pltpu.CompilerParams(use_tc_tiling_on_sc=False),
        scratch_types=[
            pltpu.VMEM_SHARED((ACC_ROWS, wb), jnp.float32),
            pltpu.VMEM((CHUNK, wb), jnp.float32),
            pltpu.VMEM((CHUNK, wb), jnp.float32),
            pltpu.VMEM((group, CHUNK), jnp.int32),
            pltpu.SemaphoreType.DMA,
        ],
    )
    def degree(sidx_hbm, out_hbm, acc, zbuf, obuf, si, sem):
        cid = lax.axis_index("c")
        sid = lax.axis_index("s")
        _fill_rows(zbuf, 0.0, wb)
        _fill_rows(obuf, 1.0, wb)
        _zero_acc(acc, zbuf, sid)
        plsc.subcore_barrier()
        chunk0 = (cid * NSUBS + sid) * chunks_per_tile

        def body(it, carry):
            base = chunk0 + it * group
            pltpu.sync_copy(sidx_hbm.at[pl.ds(base, group)], si)
            for g in range(group):
                pltpu.sync_copy(obuf, acc.at[si.at[g]], add=True)
            return carry

        lax.fori_loop(0, iters, body, 0)
        plsc.subcore_barrier()
        pltpu.sync_copy(
            acc.at[pl.ds(sid * ROWS_PER_TILE, ROWS_PER_TILE)],
            out_hbm.at[cid, pl.ds(sid * ROWS_PER_TILE, ROWS_PER_TILE)])

    return degree


def _prep_body(deg_ref, x_ref, dis_ref, s0_ref):
    d = deg_ref[0, :, 0:1] + deg_ref[1, :, 0:1]
    dis = jnp.where(d > 0, lax.rsqrt(d), 0.0)
    dis_ref[...] = jnp.broadcast_to(dis, (BLK, 64))
    s0_ref[...] = x_ref[...] * dis


@functools.lru_cache(maxsize=None)
def _make_prep():
    return pl.pallas_call(
        _prep_body,
        grid=(GRID,),
        in_specs=[
            pl.BlockSpec((2, BLK, 16), lambda i: (0, i, 0)),
            pl.BlockSpec((BLK, 16), lambda i: (i, 0)),
        ],
        out_specs=[
            pl.BlockSpec((BLK, 64), lambda i: (i, 0)),
            pl.BlockSpec((BLK, 16), lambda i: (i, 0)),
        ],
        out_shape=[
            jax.ShapeDtypeStruct((NODES_PAD, 64), jnp.float32),
            jax.ShapeDtypeStruct((NODES_PAD, 16), jnp.float32),
        ],
    )


@functools.lru_cache(maxsize=None)
def _make_combine(wb, first, feature_cat, want_s):
    wo = 2 * wb if feature_cat else wb

    def body(*refs):
        if first:
            acc_ref, dis_ref = refs[0], refs[1]
            out_refs = refs[2:]
        else:
            acc_ref, dis_ref, txp_ref = refs[0], refs[1], refs[2]
            out_refs = refs[3:]
        if feature_cat:
            a = jnp.concatenate([acc_ref[0], acc_ref[1]], axis=-1)
        else:
            a = acc_ref[0] + acc_ref[1]
        dis = dis_ref[:, :wo]
        if first:
            tx = -(dis * a)
        else:
            tx = -2.0 * (dis * a) - txp_ref[...]
        out_refs[0][...] = tx
        if want_s:
            out_refs[1][...] = dis * tx

    in_specs = [
        pl.BlockSpec((2, BLK, wb), lambda i: (0, i, 0)),
        pl.BlockSpec((BLK, 64), lambda i: (i, 0)),
    ]
    if not first:
        in_specs.append(pl.BlockSpec((BLK, wo), lambda i: (i, 0)))
    out_specs = [pl.BlockSpec((BLK, wo), lambda i: (i, 0))]
    out_shape = [jax.ShapeDtypeStruct((NODES_PAD, wo), jnp.float32)]
    if want_s:
        out_specs.append(pl.BlockSpec((BLK, wo), lambda i: (i, 0)))
        out_shape.append(jax.ShapeDtypeStruct((NODES_PAD, wo), jnp.float32))
    return pl.pallas_call(
        body, grid=(GRID,), in_specs=in_specs, out_specs=out_specs,
        out_shape=out_shape)


@functools.lru_cache(maxsize=None)
def _make_gru(nk, cin, cout, kind):
    def body(*refs):
        txs = refs[:nk]
        wz_ref, wh_ref, bz_ref, bh_ref = refs[nk:nk + 4]
        rest = refs[nk + 4:]
        az = jnp.zeros((BLK, cout), jnp.float32)
        ah = jnp.zeros((BLK, cout), jnp.float32)
        for k in range(nk):
            xk = txs[k][...]
            az = az + jnp.dot(xk, wz_ref[k], preferred_element_type=jnp.float32)
            ah = ah + jnp.dot(xk, wh_ref[k], preferred_element_type=jnp.float32)
        z = jax.nn.sigmoid(az + bz_ref[...])
        ht = jnp.tanh(ah + bh_ref[...])
        h = jax.nn.relu((1.0 - z) * ht)
        if kind == "final":
            wl_ref, bl_ref, out_ref = rest
            logits = jnp.dot(h, wl_ref[...],
                             preferred_element_type=jnp.float32) + bl_ref[...]
            out_ref[...] = jax.nn.softmax(logits, axis=-1)
        elif kind == "mid":
            dis_ref, h_ref, s_ref = rest
            h_ref[...] = h
            s_ref[...] = h * dis_ref[:, :cout]
        else:
            dis_ref, h_ref, s2_ref = rest
            h_ref[...] = h
            sv = h * dis_ref[:, :cout]
            s2_ref[0] = sv[:, :cout // 2]
            s2_ref[1] = sv[:, cout // 2:]

    in_specs = [pl.BlockSpec((BLK, cin), lambda i: (i, 0)) for _ in range(nk)]
    in_specs += [
        pl.BlockSpec((nk, cin, cout), lambda i: (0, 0, 0)),
        pl.BlockSpec((nk, cin, cout), lambda i: (0, 0, 0)),
        pl.BlockSpec((1, cout), lambda i: (0, 0)),
        pl.BlockSpec((1, cout), lambda i: (0, 0)),
    ]
    if kind == "final":
        in_specs += [
            pl.BlockSpec((cout, 2), lambda i: (0, 0)),
            pl.BlockSpec((1, 2), lambda i: (0, 0)),
        ]
        out_specs = [pl.BlockSpec((BLK, 2), lambda i: (i, 0))]
        out_shape = [jax.ShapeDtypeStruct((NODES_PAD, 2), jnp.float32)]
    elif kind == "mid":
        in_specs.append(pl.BlockSpec((BLK, 64), lambda i: (i, 0)))
        out_specs = [
            pl.BlockSpec((BLK, cout), lambda i: (i, 0)),
            pl.BlockSpec((BLK, cout), lambda i: (i, 0)),
        ]
        out_shape = [
            jax.ShapeDtypeStruct((NODES_PAD, cout), jnp.float32),
            jax.ShapeDtypeStruct((NODES_PAD, cout), jnp.float32),
        ]
    else:
        in_specs.append(pl.BlockSpec((BLK, 64), lambda i: (i, 0)))
        out_specs = [
            pl.BlockSpec((BLK, cout), lambda i: (i, 0)),
            pl.BlockSpec((2, BLK, cout // 2), lambda i: (0, i, 0)),
        ]
        out_shape = [
            jax.ShapeDtypeStruct((NODES_PAD, cout), jnp.float32),
            jax.ShapeDtypeStruct((2, NODES_PAD, cout // 2), jnp.float32),
        ]
    return pl.pallas_call(
        body, grid=(GRID,), in_specs=in_specs, out_specs=out_specs,
        out_shape=out_shape)


def _pad_w(w, cin):
    ci = w.shape[1]
    if ci == cin:
        return w
    return jnp.pad(w, ((0, 0), (0, cin - ci), (0, 0)))


def kernel(x, edge_index, params):
    row, col = edge_index[0], edge_index[1]
    npad = E_PAD - row.shape[0]
    trash = jnp.full((npad,), TRASH, jnp.int32)
    row2d = jnp.concatenate([row, trash]).reshape(N_CHUNKS, CHUNK)
    col2d = jnp.concatenate([col, trash]).reshape(N_CHUNKS, CHUNK)
    x16 = jnp.pad(x, ((0, NODES_PAD - x.shape[0]), (0, 16 - x.shape[1])))

    deg_parts = _make_degree()(row2d)
    dis64, s = _make_prep()(deg_parts, x16)

    h = x16
    for li, (cin, cout, K) in enumerate(LAYER_DIMS):
        lp = params["layers"][li]
        last_layer = li == len(LAYER_DIMS) - 1
        txs = [h]
        if last_layer:
            acc = _make_spmv(cin // 2, True)(s, row2d, col2d)
            (tx,) = _make_combine(cin // 2, True, True, False)(acc, dis64)
            txs.append(tx)
        else:
            for k in range(1, K):
                first = k == 1
                want_s = k < K - 1
                acc = _make_spmv(cin, False)(s, row2d, col2d)
                args = (acc, dis64) if first else (acc, dis64, txs[k - 2])
                outs = _make_combine(cin, first, False, want_s)(*args)
                txs.append(outs[0])
                if want_s:
                    s = outs[1]
        wz = _pad_w(lp["xz"][0], cin)
        wh = _pad_w(lp["xh"][0], cin)
        bz = (lp["xz"][1] + lp["hz"][1]).reshape(1, cout)
        bh = (lp["xh"][1] + lp["hh"][1]).reshape(1, cout)
        if last_layer:
            wl, bl = params["linear"]
            (out,) = _make_gru(K, cin, cout, "final")(
                *txs, wz, wh, bz, bh, wl, bl.reshape(1, 2))
            return out[:N_NODES]
        kind = "split" if li == len(LAYER_DIMS) - 2 else "mid"
        res = _make_gru(K, cin, cout, kind)(*txs, wz, wh, bz, bh, dis64)
        h = res[0]
        if kind == "split":
            s = res[1].reshape(2 * NODES_PAD, cout // 2)
        else:
            s = res[1]


# speedup vs baseline: 28.5100x; 1.1822x over previous
import functools

import jax
import jax.numpy as jnp
from jax import lax
from jax.experimental import pallas as pl
from jax.experimental.pallas import tpu as pltpu
from jax.experimental.pallas import tpu_sc as plsc

N_NODES = 50000
NODES_PAD = 50176
TRASH = 50048
ACC_ROWS = NODES_PAD
ROWS_PER_TILE = ACC_ROWS // 16
ZCH = 112
CHUNK = 128
E_PAD = 802816
N_CHUNKS = E_PAD // CHUNK
NCORES = 2
NSUBS = 16
BLK = 512
GRID = NODES_PAD // BLK

LAYER_DIMS = [(16, 16, 5), (16, 32, 4), (32, 64, 3), (64, 152, 2)]


def _sc_mesh():
    return plsc.VectorSubcoreMesh(core_axis_name="c", subcore_axis_name="s")


def _fill_rows(ref, value, rows, width):
    vec = jnp.full((16,), value, jnp.float32)

    def body(i, carry):
        for j in range(width // 16):
            ref[i, pl.ds(j * 16, 16)] = vec
        return carry

    lax.fori_loop(0, rows, body, 0)


def _zero_acc(acc, zbuf, sid):
    def body(t, carry):
        pltpu.sync_copy(
            zbuf, acc.at[pl.ds(sid * ROWS_PER_TILE + t * ZCH, ZCH)])
        return carry

    lax.fori_loop(0, ROWS_PER_TILE // ZCH, body, 0)


@functools.lru_cache(maxsize=None)
def _make_spmv(wb, feature_split):
    if feature_split:
        chunks_per_tile = N_CHUNKS // NSUBS
    else:
        chunks_per_tile = N_CHUNKS // (NCORES * NSUBS)
    group = 7 if wb == 16 else 2
    n_groups = chunks_per_tile // group
    assert group * n_groups == chunks_per_tile and n_groups % 2 == 0

    @functools.partial(
        pl.kernel,
        out_type=jax.ShapeDtypeStruct((NCORES, ACC_ROWS, wb), jnp.float32),
        mesh=_sc_mesh(),
        compiler_params=pltpu.CompilerParams(use_tc_tiling_on_sc=False),
        scratch_types=[
            pltpu.VMEM_SHARED((ACC_ROWS, wb), jnp.float32),
            pltpu.VMEM((ZCH, wb), jnp.float32),
            pltpu.VMEM((2, group, CHUNK), jnp.int32),
            pltpu.VMEM((2, group, CHUNK), jnp.int32),
            pltpu.VMEM((2, group, CHUNK, wb), jnp.float32),
            pltpu.SemaphoreType.DMA,
        ],
    )
    def spmv(s_tab, gidx_hbm, sidx_hbm, out_hbm, acc, zbuf, gi, si, data, sem):
        cid = lax.axis_index("c")
        sid = lax.axis_index("s")
        _fill_rows(zbuf, 0.0, ZCH, wb)
        _zero_acc(acc, zbuf, sid)
        plsc.subcore_barrier()

        if feature_split:
            chunk0 = sid * chunks_per_tile
        else:
            chunk0 = (cid * NSUBS + sid) * chunks_per_tile

        def load_idx(slot, grp):
            base = chunk0 + grp * group
            pltpu.sync_copy(gidx_hbm.at[pl.ds(base, group)], gi.at[slot])
            pltpu.sync_copy(sidx_hbm.at[pl.ds(base, group)], si.at[slot])
            if feature_split:
                off = cid * NODES_PAD
                for g in range(group):
                    for j in range(CHUNK // 16):
                        v = gi[slot, g, pl.ds(j * 16, 16)]
                        gi[slot, g, pl.ds(j * 16, 16)] = v + off

        def fire(slot):
            for g in range(group):
                pltpu.async_copy(s_tab.at[gi.at[slot, g]],
                                 data.at[slot, g], sem)

        def drain_scatter(slot):
            for g in range(group):
                pltpu.make_async_copy(s_tab.at[gi.at[slot, g]],
                                      data.at[slot, g], sem).wait()
            for g in range(group):
                pltpu.sync_copy(data.at[slot, g], acc.at[si.at[slot, g]],
                                add=True)

        load_idx(0, 0)
        fire(0)

        def body(o, carry):
            load_idx(1, 2 * o + 1)
            fire(1)
            drain_scatter(0)
            load_idx(0, lax.rem(2 * o + 2, n_groups))
            fire(0)
            drain_scatter(1)
            return carry

        lax.fori_loop(0, n_groups // 2, body, 0)
        for g in range(group):
            pltpu.make_async_copy(s_tab.at[gi.at[0, g]],
                                  data.at[0, g], sem).wait()
        plsc.subcore_barrier()
        pltpu.sync_copy(
            acc.at[pl.ds(sid * ROWS_PER_TILE, ROWS_PER_TILE)],
            out_hbm.at[cid, pl.ds(sid * ROWS_PER_TILE, ROWS_PER_TILE)])

    return spmv


@functools.lru_cache(maxsize=None)
def _make_degree():
    wb = 16
    chunks_per_tile = N_CHUNKS // (NCORES * NSUBS)
    group = 7
    n_groups = chunks_per_tile // group

    @functools.partial(
        pl.kernel,
        out_type=jax.ShapeDtypeStruct((NCORES, ACC_ROWS, wb), jnp.float32),
        mesh=_sc_mesh(),
        compiler_params=pltpu.CompilerParams(use_tc_tiling_on_sc=False),
        scratch_types=[
            pltpu.VMEM_SHARED((ACC_ROWS, wb), jnp.float32),
            pltpu.VMEM((ZCH, wb), jnp.float32),
            pltpu.VMEM((CHUNK, wb), jnp.float32),
            pltpu.VMEM((2, group, CHUNK), jnp.int32),
            pltpu.SemaphoreType.DMA,
        ],
    )
    def degree(sidx_hbm, out_hbm, acc, zbuf, obuf, si, sem):
        cid = lax.axis_index("c")
        sid = lax.axis_index("s")
        _fill_rows(zbuf, 0.0, ZCH, wb)
        _fill_rows(obuf, 1.0, CHUNK, wb)
        _zero_acc(acc, zbuf, sid)
        plsc.subcore_barrier()
        chunk0 = (cid * NSUBS + sid) * chunks_per_tile

        def load_idx(slot, grp):
            base = chunk0 + grp * group
            pltpu.sync_copy(sidx_hbm.at[pl.ds(base, group)], si.at[slot])

        def scatter(slot):
            for g in range(group):
                pltpu.sync_copy(obuf, acc.at[si.at[slot, g]], add=True)

        load_idx(0, 0)

        def body(o, carry):
            load_idx(1, 2 * o + 1)
            scatter(0)
            load_idx(0, lax.rem(2 * o + 2, n_groups))
            scatter(1)
            return carry

        lax.fori_loop(0, n_groups // 2, body, 0)
        plsc.subcore_barrier()
        pltpu.sync_copy(
            acc.at[pl.ds(sid * ROWS_PER_TILE, ROWS_PER_TILE)],
            out_hbm.at[cid, pl.ds(sid * ROWS_PER_TILE, ROWS_PER_TILE)])

    return degree


def _prep_body(deg_ref, x_ref, dis_ref, s0_ref):
    d = deg_ref[0, :, 0:1] + deg_ref[1, :, 0:1]
    dis = jnp.where(d > 0, lax.rsqrt(d), 0.0)
    dis_ref[...] = jnp.broadcast_to(dis, (BLK, 64))
    s0_ref[...] = x_ref[...] * dis


@functools.lru_cache(maxsize=None)
def _make_prep():
    return pl.pallas_call(
        _prep_body,
        grid=(GRID,),
        in_specs=[
            pl.BlockSpec((2, BLK, 16), lambda i: (0, i, 0)),
            pl.BlockSpec((BLK, 16), lambda i: (i, 0)),
        ],
        out_specs=[
            pl.BlockSpec((BLK, 64), lambda i: (i, 0)),
            pl.BlockSpec((BLK, 16), lambda i: (i, 0)),
        ],
        out_shape=[
            jax.ShapeDtypeStruct((NODES_PAD, 64), jnp.float32),
            jax.ShapeDtypeStruct((NODES_PAD, 16), jnp.float32),
        ],
    )


@functools.lru_cache(maxsize=None)
def _make_combine(wb, first, feature_cat, want_s):
    wo = 2 * wb if feature_cat else wb

    def body(*refs):
        if first:
            acc_ref, dis_ref = refs[0], refs[1]
            out_refs = refs[2:]
        else:
            acc_ref, dis_ref, txp_ref = refs[0], refs[1], refs[2]
            out_refs = refs[3:]
        if feature_cat:
            a = jnp.concatenate([acc_ref[0], acc_ref[1]], axis=-1)
        else:
            a = acc_ref[0] + acc_ref[1]
        dis = dis_ref[:, :wo]
        if first:
            tx = -(dis * a)
        else:
            tx = -2.0 * (dis * a) - txp_ref[...]
        out_refs[0][...] = tx
        if want_s:
            out_refs[1][...] = dis * tx

    in_specs = [
        pl.BlockSpec((2, BLK, wb), lambda i: (0, i, 0)),
        pl.BlockSpec((BLK, 64), lambda i: (i, 0)),
    ]
    if not first:
        in_specs.append(pl.BlockSpec((BLK, wo), lambda i: (i, 0)))
    out_specs = [pl.BlockSpec((BLK, wo), lambda i: (i, 0))]
    out_shape = [jax.ShapeDtypeStruct((NODES_PAD, wo), jnp.float32)]
    if want_s:
        out_specs.append(pl.BlockSpec((BLK, wo), lambda i: (i, 0)))
        out_shape.append(jax.ShapeDtypeStruct((NODES_PAD, wo), jnp.float32))
    return pl.pallas_call(
        body, grid=(GRID,), in_specs=in_specs, out_specs=out_specs,
        out_shape=out_shape)


@functools.lru_cache(maxsize=None)
def _make_gru(nk, cin, cout, kind):
    def body(*refs):
        txs = refs[:nk]
        wz_ref, wh_ref, bz_ref, bh_ref = refs[nk:nk + 4]
        rest = refs[nk + 4:]
        az = jnp.zeros((BLK, cout), jnp.float32)
        ah = jnp.zeros((BLK, cout), jnp.float32)
        for k in range(nk):
            xk = txs[k][...]
            az = az + jnp.dot(xk, wz_ref[k], preferred_element_type=jnp.float32)
            ah = ah + jnp.dot(xk, wh_ref[k], preferred_element_type=jnp.float32)
        z = jax.nn.sigmoid(az + bz_ref[...])
        ht = jnp.tanh(ah + bh_ref[...])
        h = jax.nn.relu((1.0 - z) * ht)
        if kind == "final":
            wl_ref, bl_ref, out_ref = rest
            logits = jnp.dot(h, wl_ref[...],
                             preferred_element_type=jnp.float32) + bl_ref[...]
            out_ref[...] = jax.nn.softmax(logits, axis=-1)
        elif kind == "mid":
            dis_ref, h_ref, s_ref = rest
            h_ref[...] = h
            s_ref[...] = h * dis_ref[:, :cout]
        else:
            dis_ref, h_ref, s2_ref = rest
            h_ref[...] = h
            sv = h * dis_ref[:, :cout]
            s2_ref[0] = sv[:, :cout // 2]
            s2_ref[1] = sv[:, cout // 2:]

    in_specs = [pl.BlockSpec((BLK, cin), lambda i: (i, 0)) for _ in range(nk)]
    in_specs += [
        pl.BlockSpec((nk, cin, cout), lambda i: (0, 0, 0)),
        pl.BlockSpec((nk, cin, cout), lambda i: (0, 0, 0)),
        pl.BlockSpec((1, cout), lambda i: (0, 0)),
        pl.BlockSpec((1, cout), lambda i: (0, 0)),
    ]
    if kind == "final":
        in_specs += [
            pl.BlockSpec((cout, 2), lambda i: (0, 0)),
            pl.BlockSpec((1, 2), lambda i: (0, 0)),
        ]
        out_specs = [pl.BlockSpec((BLK, 2), lambda i: (i, 0))]
        out_shape = [jax.ShapeDtypeStruct((NODES_PAD, 2), jnp.float32)]
    elif kind == "mid":
        in_specs.append(pl.BlockSpec((BLK, 64), lambda i: (i, 0)))
        out_specs = [
            pl.BlockSpec((BLK, cout), lambda i: (i, 0)),
            pl.BlockSpec((BLK, cout), lambda i: (i, 0)),
        ]
        out_shape = [
            jax.ShapeDtypeStruct((NODES_PAD, cout), jnp.float32),
            jax.ShapeDtypeStruct((NODES_PAD, cout), jnp.float32),
        ]
    else:
        in_specs.append(pl.BlockSpec((BLK, 64), lambda i: (i, 0)))
        out_specs = [
            pl.BlockSpec((BLK, cout), lambda i: (i, 0)),
            pl.BlockSpec((2, BLK, cout // 2), lambda i: (0, i, 0)),
        ]
        out_shape = [
            jax.ShapeDtypeStruct((NODES_PAD, cout), jnp.float32),
            jax.ShapeDtypeStruct((2, NODES_PAD, cout // 2), jnp.float32),
        ]
    return pl.pallas_call(
        body, grid=(GRID,), in_specs=in_specs, out_specs=out_specs,
        out_shape=out_shape)


def _pad_w(w, cin):
    ci = w.shape[1]
    if ci == cin:
        return w
    return jnp.pad(w, ((0, 0), (0, cin - ci), (0, 0)))


def kernel(x, edge_index, params):
    row, col = edge_index[0], edge_index[1]
    npad = E_PAD - row.shape[0]
    trash = jnp.full((npad,), TRASH, jnp.int32)
    row2d = jnp.concatenate([row, trash]).reshape(N_CHUNKS, CHUNK)
    col2d = jnp.concatenate([col, trash]).reshape(N_CHUNKS, CHUNK)
    x16 = jnp.pad(x, ((0, NODES_PAD - x.shape[0]), (0, 16 - x.shape[1])))

    deg_parts = _make_degree()(row2d)
    dis64, s = _make_prep()(deg_parts, x16)

    h = x16
    for li, (cin, cout, K) in enumerate(LAYER_DIMS):
        lp = params["layers"][li]
        last_layer = li == len(LAYER_DIMS) - 1
        txs = [h]
        if last_layer:
            acc = _make_spmv(cin // 2, True)(s, row2d, col2d)
            (tx,) = _make_combine(cin // 2, True, True, False)(acc, dis64)
            txs.append(tx)
        else:
            for k in range(1, K):
                first = k == 1
                want_s = k < K - 1
                acc = _make_spmv(cin, False)(s, row2d, col2d)
                args = (acc, dis64) if first else (acc, dis64, txs[k - 2])
                outs = _make_combine(cin, first, False, want_s)(*args)
                txs.append(outs[0])
                if want_s:
                    s = outs[1]
        wz = _pad_w(lp["xz"][0], cin)
        wh = _pad_w(lp["xh"][0], cin)
        bz = (lp["xz"][1] + lp["hz"][1]).reshape(1, cout)
        bh = (lp["xh"][1] + lp["hh"][1]).reshape(1, cout)
        if last_layer:
            wl, bl = params["linear"]
            (out,) = _make_gru(K, cin, cout, "final")(
                *txs, wz, wh, bz, bh, wl, bl.reshape(1, 2))
            return out[:N_NODES]
        kind = "split" if li == len(LAYER_DIMS) - 2 else "mid"
        res = _make_gru(K, cin, cout, kind)(*txs, wz, wh, bz, bh, dis64)
        h = res[0]
        if kind == "split":
            s = res[1].reshape(2 * NODES_PAD, cout // 2)
        else:
            s = res[1]


# speedup vs baseline: 29.5526x; 1.0366x over previous
import functools

import jax
import jax.numpy as jnp
from jax import lax
from jax.experimental import pallas as pl
from jax.experimental.pallas import tpu as pltpu
from jax.experimental.pallas import tpu_sc as plsc

N_NODES = 50000
NODES_PAD = 50176
TRASH = 50048
ACC_ROWS = NODES_PAD
ROWS_PER_TILE = ACC_ROWS // 16
ZCH = 112
CHUNK = 128
E_PAD = 802816
N_CHUNKS = E_PAD // CHUNK
NCORES = 2
NSUBS = 16
BLK = 512
GRID = NODES_PAD // BLK

LAYER_DIMS = [(16, 16, 5), (16, 32, 4), (32, 64, 3), (64, 152, 2)]


def _sc_mesh():
    return plsc.VectorSubcoreMesh(core_axis_name="c", subcore_axis_name="s")


def _fill_rows(ref, value, rows, width):
    vec = jnp.full((16,), value, jnp.float32)

    def body(i, carry):
        for j in range(width // 16):
            ref[i, pl.ds(j * 16, 16)] = vec
        return carry

    lax.fori_loop(0, rows, body, 0)


def _zero_acc(acc, zbuf, sid):
    def body(t, carry):
        pltpu.sync_copy(
            zbuf, acc.at[pl.ds(sid * ROWS_PER_TILE + t * ZCH, ZCH)])
        return carry

    lax.fori_loop(0, ROWS_PER_TILE // ZCH, body, 0)


@functools.lru_cache(maxsize=None)
def _make_spmv(wb, feature_split):
    if feature_split:
        chunks_per_tile = N_CHUNKS // NSUBS
    else:
        chunks_per_tile = N_CHUNKS // (NCORES * NSUBS)
    group = 7 if wb == 16 else 2
    n_groups = chunks_per_tile // group
    assert group * n_groups == chunks_per_tile and n_groups % 2 == 0

    @functools.partial(
        pl.kernel,
        out_type=jax.ShapeDtypeStruct((NCORES, ACC_ROWS, wb), jnp.float32),
        mesh=_sc_mesh(),
        compiler_params=pltpu.CompilerParams(use_tc_tiling_on_sc=False),
        scratch_types=[
            pltpu.VMEM_SHARED((ACC_ROWS, wb), jnp.float32),
            pltpu.VMEM((ZCH, wb), jnp.float32),
            pltpu.VMEM((2, group, CHUNK), jnp.int32),
            pltpu.VMEM((2, group, CHUNK), jnp.int32),
            pltpu.VMEM((2, group, CHUNK, wb), jnp.float32),
            pltpu.SemaphoreType.DMA,
            pltpu.SemaphoreType.DMA,
        ],
    )
    def spmv(s_tab, gidx_hbm, sidx_hbm, out_hbm, acc, zbuf, gi, si, data, sem,
             ssem):
        cid = lax.axis_index("c")
        sid = lax.axis_index("s")
        _fill_rows(zbuf, 0.0, ZCH, wb)
        _zero_acc(acc, zbuf, sid)
        plsc.subcore_barrier()

        if feature_split:
            chunk0 = sid * chunks_per_tile
        else:
            chunk0 = (cid * NSUBS + sid) * chunks_per_tile

        def load_idx(slot, grp):
            base = chunk0 + grp * group
            pltpu.sync_copy(gidx_hbm.at[pl.ds(base, group)], gi.at[slot])
            pltpu.sync_copy(sidx_hbm.at[pl.ds(base, group)], si.at[slot])
            if feature_split:
                off = cid * NODES_PAD
                for g in range(group):
                    for j in range(CHUNK // 16):
                        v = gi[slot, g, pl.ds(j * 16, 16)]
                        gi[slot, g, pl.ds(j * 16, 16)] = v + off

        def fire(slot):
            for g in range(group):
                pltpu.async_copy(s_tab.at[gi.at[slot, g]],
                                 data.at[slot, g], sem)

        def drain_gather(slot):
            for g in range(group):
                pltpu.make_async_copy(s_tab.at[gi.at[slot, g]],
                                      data.at[slot, g], sem).wait()

        def fire_scatter(slot):
            for g in range(group):
                pltpu.async_copy(data.at[slot, g], acc.at[si.at[slot, g]],
                                 ssem, add=True)

        def wait_scatter(slot):
            for g in range(group):
                pltpu.make_async_copy(data.at[slot, g],
                                      acc.at[si.at[slot, g]], ssem).wait()

        load_idx(0, 0)
        fire(0)

        def body(o, carry):
            load_idx(1, 2 * o + 1)
            fire(1)
            drain_gather(0)
            fire_scatter(0)
            load_idx(0, lax.rem(2 * o + 2, n_groups))
            wait_scatter(0)
            fire(0)
            drain_gather(1)
            fire_scatter(1)
            wait_scatter(1)
            return carry

        lax.fori_loop(0, n_groups // 2, body, 0)
        for g in range(group):
            pltpu.make_async_copy(s_tab.at[gi.at[0, g]],
                                  data.at[0, g], sem).wait()
        plsc.subcore_barrier()
        pltpu.sync_copy(
            acc.at[pl.ds(sid * ROWS_PER_TILE, ROWS_PER_TILE)],
            out_hbm.at[cid, pl.ds(sid * ROWS_PER_TILE, ROWS_PER_TILE)])

    return spmv


@functools.lru_cache(maxsize=None)
def _make_degree():
    wb = 16
    chunks_per_tile = N_CHUNKS // (NCORES * NSUBS)
    group = 7
    n_groups = chunks_per_tile // group

    @functools.partial(
        pl.kernel,
        out_type=jax.ShapeDtypeStruct((NCORES, ACC_ROWS, wb), jnp.float32),
        mesh=_sc_mesh(),
        compiler_params=pltpu.CompilerParams(use_tc_tiling_on_sc=False),
        scratch_types=[
            pltpu.VMEM_SHARED((ACC_ROWS, wb), jnp.float32),
            pltpu.VMEM((ZCH, wb), jnp.float32),
            pltpu.VMEM((CHUNK, wb), jnp.float32),
            pltpu.VMEM((2, group, CHUNK), jnp.int32),
            pltpu.SemaphoreType.DMA,
        ],
    )
    def degree(sidx_hbm, out_hbm, acc, zbuf, obuf, si, sem):
        cid = lax.axis_index("c")
        sid = lax.axis_index("s")
        _fill_rows(zbuf, 0.0, ZCH, wb)
        _fill_rows(obuf, 1.0, CHUNK, wb)
        _zero_acc(acc, zbuf, sid)
        plsc.subcore_barrier()
        chunk0 = (cid * NSUBS + sid) * chunks_per_tile

        def load_idx(slot, grp):
            base = chunk0 + grp * group
            pltpu.sync_copy(sidx_hbm.at[pl.ds(base, group)], si.at[slot])

        def scatter(slot):
            for g in range(group):
                pltpu.async_copy(obuf, acc.at[si.at[slot, g]], sem, add=True)
            for g in range(group):
                pltpu.make_async_copy(obuf, acc.at[si.at[slot, g]],
                                      sem).wait()

        load_idx(0, 0)

        def body(o, carry):
            load_idx(1, 2 * o + 1)
            scatter(0)
            load_idx(0, lax.rem(2 * o + 2, n_groups))
            scatter(1)
            return carry

        lax.fori_loop(0, n_groups // 2, body, 0)
        plsc.subcore_barrier()
        pltpu.sync_copy(
            acc.at[pl.ds(sid * ROWS_PER_TILE, ROWS_PER_TILE)],
            out_hbm.at[cid, pl.ds(sid * ROWS_PER_TILE, ROWS_PER_TILE)])

    return degree


def _prep_body(deg_ref, x_ref, dis_ref, s0_ref):
    d = deg_ref[0, :, 0:1] + deg_ref[1, :, 0:1]
    dis = jnp.where(d > 0, lax.rsqrt(d), 0.0)
    dis_ref[...] = jnp.broadcast_to(dis, (BLK, 64))
    s0_ref[...] = x_ref[...] * dis


@functools.lru_cache(maxsize=None)
def _make_prep():
    return pl.pallas_call(
        _prep_body,
        grid=(GRID,),
        in_specs=[
            pl.BlockSpec((2, BLK, 16), lambda i: (0, i, 0)),
            pl.BlockSpec((BLK, 16), lambda i: (i, 0)),
        ],
        out_specs=[
            pl.BlockSpec((BLK, 64), lambda i: (i, 0)),
            pl.BlockSpec((BLK, 16), lambda i: (i, 0)),
        ],
        out_shape=[
            jax.ShapeDtypeStruct((NODES_PAD, 64), jnp.float32),
            jax.ShapeDtypeStruct((NODES_PAD, 16), jnp.float32),
        ],
    )


@functools.lru_cache(maxsize=None)
def _make_combine(wb, first, feature_cat, want_s):
    wo = 2 * wb if feature_cat else wb

    def body(*refs):
        if first:
            acc_ref, dis_ref = refs[0], refs[1]
            out_refs = refs[2:]
        else:
            acc_ref, dis_ref, txp_ref = refs[0], refs[1], refs[2]
            out_refs = refs[3:]
        if feature_cat:
            a = jnp.concatenate([acc_ref[0], acc_ref[1]], axis=-1)
        else:
            a = acc_ref[0] + acc_ref[1]
        dis = dis_ref[:, :wo]
        if first:
            tx = -(dis * a)
        else:
            tx = -2.0 * (dis * a) - txp_ref[...]
        out_refs[0][...] = tx
        if want_s:
            out_refs[1][...] = dis * tx

    in_specs = [
        pl.BlockSpec((2, BLK, wb), lambda i: (0, i, 0)),
        pl.BlockSpec((BLK, 64), lambda i: (i, 0)),
    ]
    if not first:
        in_specs.append(pl.BlockSpec((BLK, wo), lambda i: (i, 0)))
    out_specs = [pl.BlockSpec((BLK, wo), lambda i: (i, 0))]
    out_shape = [jax.ShapeDtypeStruct((NODES_PAD, wo), jnp.float32)]
    if want_s:
        out_specs.append(pl.BlockSpec((BLK, wo), lambda i: (i, 0)))
        out_shape.append(jax.ShapeDtypeStruct((NODES_PAD, wo), jnp.float32))
    return pl.pallas_call(
        body, grid=(GRID,), in_specs=in_specs, out_specs=out_specs,
        out_shape=out_shape)


@functools.lru_cache(maxsize=None)
def _make_gru(nk, cin, cout, kind):
    def body(*refs):
        txs = refs[:nk]
        wz_ref, wh_ref, bz_ref, bh_ref = refs[nk:nk + 4]
        rest = refs[nk + 4:]
        az = jnp.zeros((BLK, cout), jnp.float32)
        ah = jnp.zeros((BLK, cout), jnp.float32)
        for k in range(nk):
            xk = txs[k][...]
            az = az + jnp.dot(xk, wz_ref[k], preferred_element_type=jnp.float32)
            ah = ah + jnp.dot(xk, wh_ref[k], preferred_element_type=jnp.float32)
        z = jax.nn.sigmoid(az + bz_ref[...])
        ht = jnp.tanh(ah + bh_ref[...])
        h = jax.nn.relu((1.0 - z) * ht)
        if kind == "final":
            wl_ref, bl_ref, out_ref = rest
            logits = jnp.dot(h, wl_ref[...],
                             preferred_element_type=jnp.float32) + bl_ref[...]
            out_ref[...] = jax.nn.softmax(logits, axis=-1)
        elif kind == "mid":
            dis_ref, h_ref, s_ref = rest
            h_ref[...] = h
            s_ref[...] = h * dis_ref[:, :cout]
        else:
            dis_ref, h_ref, s2_ref = rest
            h_ref[...] = h
            sv = h * dis_ref[:, :cout]
            s2_ref[0] = sv[:, :cout // 2]
            s2_ref[1] = sv[:, cout // 2:]

    in_specs = [pl.BlockSpec((BLK, cin), lambda i: (i, 0)) for _ in range(nk)]
    in_specs += [
        pl.BlockSpec((nk, cin, cout), lambda i: (0, 0, 0)),
        pl.BlockSpec((nk, cin, cout), lambda i: (0, 0, 0)),
        pl.BlockSpec((1, cout), lambda i: (0, 0)),
        pl.BlockSpec((1, cout), lambda i: (0, 0)),
    ]
    if kind == "final":
        in_specs += [
            pl.BlockSpec((cout, 2), lambda i: (0, 0)),
            pl.BlockSpec((1, 2), lambda i: (0, 0)),
        ]
        out_specs = [pl.BlockSpec((BLK, 2), lambda i: (i, 0))]
        out_shape = [jax.ShapeDtypeStruct((NODES_PAD, 2), jnp.float32)]
    elif kind == "mid":
        in_specs.append(pl.BlockSpec((BLK, 64), lambda i: (i, 0)))
        out_specs = [
            pl.BlockSpec((BLK, cout), lambda i: (i, 0)),
            pl.BlockSpec((BLK, cout), lambda i: (i, 0)),
        ]
        out_shape = [
            jax.ShapeDtypeStruct((NODES_PAD, cout), jnp.float32),
            jax.ShapeDtypeStruct((NODES_PAD, cout), jnp.float32),
        ]
    else:
        in_specs.append(pl.BlockSpec((BLK, 64), lambda i: (i, 0)))
        out_specs = [
            pl.BlockSpec((BLK, cout), lambda i: (i, 0)),
            pl.BlockSpec((2, BLK, cout // 2), lambda i: (0, i, 0)),
        ]
        out_shape = [
            jax.ShapeDtypeStruct((NODES_PAD, cout), jnp.float32),
            jax.ShapeDtypeStruct((2, NODES_PAD, cout // 2), jnp.float32),
        ]
    return pl.pallas_call(
        body, grid=(GRID,), in_specs=in_specs, out_specs=out_specs,
        out_shape=out_shape)


def _pad_w(w, cin):
    ci = w.shape[1]
    if ci == cin:
        return w
    return jnp.pad(w, ((0, 0), (0, cin - ci), (0, 0)))


def kernel(x, edge_index, params):
    row, col = edge_index[0], edge_index[1]
    npad = E_PAD - row.shape[0]
    trash = jnp.full((npad,), TRASH, jnp.int32)
    row2d = jnp.concatenate([row, trash]).reshape(N_CHUNKS, CHUNK)
    col2d = jnp.concatenate([col, trash]).reshape(N_CHUNKS, CHUNK)
    x16 = jnp.pad(x, ((0, NODES_PAD - x.shape[0]), (0, 16 - x.shape[1])))

    deg_parts = _make_degree()(row2d)
    dis64, s = _make_prep()(deg_parts, x16)

    h = x16
    for li, (cin, cout, K) in enumerate(LAYER_DIMS):
        lp = params["layers"][li]
        last_layer = li == len(LAYER_DIMS) - 1
        txs = [h]
        if last_layer:
            acc = _make_spmv(cin // 2, True)(s, row2d, col2d)
            (tx,) = _make_combine(cin // 2, True, True, False)(acc, dis64)
            txs.append(tx)
        else:
            for k in range(1, K):
                first = k == 1
                want_s = k < K - 1
                acc = _make_spmv(cin, False)(s, row2d, col2d)
                args = (acc, dis64) if first else (acc, dis64, txs[k - 2])
                outs = _make_combine(cin, first, False, want_s)(*args)
                txs.append(outs[0])
                if want_s:
                    s = outs[1]
        wz = _pad_w(lp["xz"][0], cin)
        wh = _pad_w(lp["xh"][0], cin)
        bz = (lp["xz"][1] + lp["hz"][1]).reshape(1, cout)
        bh = (lp["xh"][1] + lp["hh"][1]).reshape(1, cout)
        if last_layer:
            wl, bl = params["linear"]
            (out,) = _make_gru(K, cin, cout, "final")(
                *txs, wz, wh, bz, bh, wl, bl.reshape(1, 2))
            return out[:N_NODES]
        kind = "split" if li == len(LAYER_DIMS) - 2 else "mid"
        res = _make_gru(K, cin, cout, kind)(*txs, wz, wh, bz, bh, dis64)
        h = res[0]
        if kind == "split":
            s = res[1].reshape(2 * NODES_PAD, cout // 2)
        else:
            s = res[1]
